# baseline (speedup 1.0000x reference)
import functools

import jax
import jax.numpy as jnp
from jax import lax
from jax.experimental import pallas as pl
from jax.experimental.pallas import tpu as pltpu
from jax.experimental.pallas import tpu_sc as plsc

_NC = 2
_NS = 16
_NW = _NC * _NS
_L = 16
_L2 = 32
_C = 80


def _sc_edge_dot(h16, ei):
    V, D = h16.shape
    E = ei.shape[0] // 2 * ei.shape[1]
    n_chunks = E // _NW // _C
    kd = D // _L2

    mesh = plsc.VectorSubcoreMesh(core_axis_name="c", subcore_axis_name="s")

    @functools.partial(
        pl.kernel,
        mesh=mesh,
        out_type=jax.ShapeDtypeStruct((E,), jnp.float32),
        scratch_types=[
            pltpu.VMEM((E // _NW,), jnp.int32),
            pltpu.VMEM((E // _NW,), jnp.int32),
            pltpu.VMEM((_C, D), jnp.bfloat16),
            pltpu.VMEM((_C, D), jnp.bfloat16),
            pltpu.VMEM((_C, D), jnp.bfloat16),
            pltpu.VMEM((_C, D), jnp.bfloat16),
            pltpu.VMEM((_C * _L,), jnp.float32),
            pltpu.VMEM((E // _NW,), jnp.float32),
            pltpu.SemaphoreType.DMA,
            pltpu.SemaphoreType.DMA,
            pltpu.SemaphoreType.DMA,
            pltpu.SemaphoreType.DMA,
        ],
        compiler_params=pltpu.CompilerParams(
            needs_layout_passes=False, use_tc_tiling_on_sc=False),
    )
    def body(h_hbm, ei_hbm, out_hbm,
             sidx, didx, srows0, srows1, drows0, drows1, tsc, obuf,
             sem_s0, sem_s1, sem_d0, sem_d1):
        epw = E // _NW
        wid = lax.axis_index("s") * _NC + lax.axis_index("c")
        base_e = pl.multiple_of(wid * epw, 8)
        pltpu.sync_copy(ei_hbm.at[wid], sidx)
        pltpu.sync_copy(ei_hbm.at[_NW + wid], didx)

        lane_ids = lax.iota(jnp.int32, _L)

        def idx_at(buf, t):
            return buf.at[pl.ds(pl.multiple_of(t * _C, _L), _C)]

        pltpu.async_copy(h_hbm.at[idx_at(sidx, 0)], srows0, sem_s0)
        pltpu.async_copy(h_hbm.at[idx_at(didx, 0)], drows0, sem_d0)
        pltpu.async_copy(h_hbm.at[idx_at(sidx, 1)], srows1, sem_s1)
        pltpu.async_copy(h_hbm.at[idx_at(didx, 1)], drows1, sem_d1)

        def process(t, srows, drows, sem_s, sem_d):
            pltpu.make_async_copy(h_hbm.at[idx_at(sidx, t)], srows, sem_s).wait()
            pltpu.make_async_copy(h_hbm.at[idx_at(didx, t)], drows, sem_d).wait()

            def main_phase(g):
                row0 = g * _L
                for eb in range(0, _L, 8):
                    acc0 = [None] * 8
                    acc1 = [None] * 8
                    for k in range(kd):
                        for e in range(8):
                            row = row0 + eb + e
                            ps = srows[row, pl.ds(k * _L2, _L2)]
                            pd = drows[row, pl.ds(k * _L2, _L2)]
                            p = ps * pd
                            if k % 2 == 0:
                                acc0[e] = p if k == 0 else acc0[e] + p
                            else:
                                acc1[e] = p if k == 1 else acc1[e] + p
                    for e in range(8):
                        tot_e = (acc0[e] if acc1[e] is None
                                 else acc0[e] + acc1[e])
                        lo, hi = plsc.unpack(
                            tot_e, format=plsc.PackFormat.INTERLEAVED)
                        tsc[pl.ds((row0 + eb + e) * _L, _L)] = lo + hi

            def reduce_phase(g):
                gbase = g * _L * _L
                cols = [plsc.load_gather(tsc, [lane_ids * _L + gbase + j])
                        for j in range(_L)]
                while len(cols) > 1:
                    cols = [cols[i] + cols[i + 1]
                            for i in range(0, len(cols), 2)]
                off = pl.multiple_of(t * _C + g * _L, _L)
                obuf[pl.ds(off, _L)] = cols[0]

            @plsc.parallel_loop(0, _C // _L)
            def group_body(g):
                main_phase(g)
                reduce_phase(g)

            @pl.when(t + 2 < n_chunks)
            def _():
                pltpu.async_copy(h_hbm.at[idx_at(sidx, t + 2)], srows, sem_s)
                pltpu.async_copy(h_hbm.at[idx_at(didx, t + 2)], drows, sem_d)

        def chunk_body(t, carry):
            @pl.when(t % 2 == 0)
            def _():
                process(t, srows0, drows0, sem_s0, sem_d0)

            @pl.when(t % 2 == 1)
            def _():
                process(t, srows1, drows1, sem_s1, sem_d1)

            return carry

        lax.fori_loop(0, n_chunks, chunk_body, 0, unroll=False)
        pltpu.sync_copy(obuf, out_hbm.at[pl.ds(base_e, epw)])

    return body(h16, ei)


def kernel(h, edge_index):
    E = edge_index.shape[1]
    h16 = h.astype(jnp.bfloat16)
    ei = edge_index.astype(jnp.int32).reshape(2 * _NW, E // _NW)
    return _sc_edge_dot(h16, ei).reshape(E, 1)


# speedup vs baseline: 1.1751x; 1.1751x over previous
import functools

import jax
import jax.numpy as jnp
from jax import lax
from jax.experimental import pallas as pl
from jax.experimental.pallas import tpu as pltpu
from jax.experimental.pallas import tpu_sc as plsc

_NC = 2
_NS = 16
_NW = _NC * _NS
_L = 16
_L2 = 32
_C = 80


def _sc_edge_dot(h16, ei):
    V, D = h16.shape
    E = ei.shape[0] // 2 * ei.shape[1]
    n_chunks = E // _NW // _C
    kd = D // _L2

    mesh = plsc.VectorSubcoreMesh(core_axis_name="c", subcore_axis_name="s")

    @functools.partial(
        pl.kernel,
        mesh=mesh,
        out_type=jax.ShapeDtypeStruct((E,), jnp.float32),
        scratch_types=[
            pltpu.VMEM((E // _NW,), jnp.int32),
            pltpu.VMEM((E // _NW,), jnp.int32),
            pltpu.VMEM((_C, D), jnp.bfloat16),
            pltpu.VMEM((_C, D), jnp.bfloat16),
            pltpu.VMEM((_C, D), jnp.bfloat16),
            pltpu.VMEM((_C, D), jnp.bfloat16),
            pltpu.VMEM((_C * _L,), jnp.float32),
            pltpu.VMEM((E // _NW,), jnp.float32),
            pltpu.SemaphoreType.DMA,
            pltpu.SemaphoreType.DMA,
            pltpu.SemaphoreType.DMA,
            pltpu.SemaphoreType.DMA,
        ],
        compiler_params=pltpu.CompilerParams(
            needs_layout_passes=False, use_tc_tiling_on_sc=False),
    )
    def body(h_hbm, ei_hbm, out_hbm,
             sidx, didx, srows0, srows1, drows0, drows1, tsc, obuf,
             sem_s0, sem_s1, sem_d0, sem_d1):
        epw = E // _NW
        wid = lax.axis_index("s") * _NC + lax.axis_index("c")
        base_e = pl.multiple_of(wid * epw, 8)
        pltpu.sync_copy(ei_hbm.at[wid], sidx)
        pltpu.sync_copy(ei_hbm.at[_NW + wid], didx)

        lane_ids = lax.iota(jnp.int32, _L)

        def idx_at(buf, t):
            return buf.at[pl.ds(pl.multiple_of(t * _C, _L), _C)]

        pltpu.async_copy(h_hbm.at[idx_at(sidx, 0)], srows0, sem_s0)
        pltpu.async_copy(h_hbm.at[idx_at(didx, 0)], drows0, sem_d0)
        pltpu.async_copy(h_hbm.at[idx_at(sidx, 1)], srows1, sem_s1)
        pltpu.async_copy(h_hbm.at[idx_at(didx, 1)], drows1, sem_d1)

        def process(t, srows, drows, sem_s, sem_d):
            pltpu.make_async_copy(h_hbm.at[idx_at(sidx, t)], srows, sem_s).wait()
            pltpu.make_async_copy(h_hbm.at[idx_at(didx, t)], drows, sem_d).wait()

            def main_phase(g):
                row0 = g * _L
                for eb in range(0, _L, 8):
                    acc0 = [None] * 8
                    acc1 = [None] * 8
                    for k in range(kd):
                        for e in range(8):
                            row = row0 + eb + e
                            ps = srows[row, pl.ds(k * _L2, _L2)]
                            pd = drows[row, pl.ds(k * _L2, _L2)]
                            p = ps * pd
                            if k % 2 == 0:
                                acc0[e] = p if k == 0 else acc0[e] + p
                            else:
                                acc1[e] = p if k == 1 else acc1[e] + p
                    for e in range(8):
                        tot_e = (acc0[e] if acc1[e] is None
                                 else acc0[e] + acc1[e])
                        lo, hi = plsc.unpack(
                            tot_e, format=plsc.PackFormat.INTERLEAVED)
                        tsc[pl.ds((row0 + eb + e) * _L, _L)] = lo + hi

            def reduce_phase(g):
                gbase = g * _L * _L
                cols = [plsc.load_gather(tsc, [lane_ids * _L + gbase + j])
                        for j in range(_L)]
                while len(cols) > 1:
                    cols = [cols[i] + cols[i + 1]
                            for i in range(0, len(cols), 2)]
                off = pl.multiple_of(t * _C + g * _L, _L)
                obuf[pl.ds(off, _L)] = cols[0]

            def group_body(g, carry2):
                main_phase(g)
                reduce_phase(g)
                return carry2

            lax.fori_loop(0, _C // _L, group_body, 0, unroll=False)

            @pl.when(t + 2 < n_chunks)
            def _():
                pltpu.async_copy(h_hbm.at[idx_at(sidx, t + 2)], srows, sem_s)
                pltpu.async_copy(h_hbm.at[idx_at(didx, t + 2)], drows, sem_d)

        def chunk_body(t, carry):
            @pl.when(t % 2 == 0)
            def _():
                process(t, srows0, drows0, sem_s0, sem_d0)

            @pl.when(t % 2 == 1)
            def _():
                process(t, srows1, drows1, sem_s1, sem_d1)

            return carry

        lax.fori_loop(0, n_chunks, chunk_body, 0, unroll=False)
        pltpu.sync_copy(obuf, out_hbm.at[pl.ds(base_e, epw)])

    return body(h16, ei)


def kernel(h, edge_index):
    E = edge_index.shape[1]
    h16 = h.astype(jnp.bfloat16)
    ei = edge_index.astype(jnp.int32).reshape(2 * _NW, E // _NW)
    return _sc_edge_dot(h16, ei).reshape(E, 1)
